# baseline (speedup 1.0000x reference)
import jax
import jax.numpy as jnp
from jax.experimental import pallas as pl


def _permute_body(x_ref, p_ref, o_ref):
    o_ref[...] = jnp.dot(x_ref[...], p_ref[...],
                         precision=jax.lax.Precision.HIGHEST,
                         preferred_element_type=o_ref.dtype)


def kernel(x, salient_channels):
    B, C, H, W = x.shape
    CO = 4 * C
    N = B * H * W
    R = 512
    while N % R:
        R //= 2

    xt = jnp.transpose(x, (0, 2, 3, 1)).reshape(N, C)
    P = jax.nn.one_hot(salient_channels, CO, dtype=x.dtype)

    out2 = pl.pallas_call(
        _permute_body,
        grid=(N // R,),
        in_specs=[
            pl.BlockSpec((R, C), lambda i: (i, 0)),
            pl.BlockSpec((C, CO), lambda i: (0, 0)),
        ],
        out_specs=pl.BlockSpec((R, CO), lambda i: (i, 0)),
        out_shape=jax.ShapeDtypeStruct((N, CO), x.dtype),
    )(xt, P)
    return out2.reshape(B, H, W, CO).transpose(0, 3, 1, 2)


# speedup vs baseline: 1.3535x; 1.3535x over previous
import jax
import jax.numpy as jnp
from jax.experimental import pallas as pl


def _permute_body(x_ref, p_ref, o_ref):
    xv = x_ref[...]
    pv = p_ref[...]
    x0 = xv.astype(jnp.bfloat16).astype(jnp.float32)
    r1 = xv - x0
    x1 = r1.astype(jnp.bfloat16).astype(jnp.float32)
    x2 = r1 - x1

    def mm(a):
        return jnp.dot(a, pv, preferred_element_type=jnp.float32)

    o_ref[...] = mm(x0) + mm(x1) + mm(x2)


def kernel(x, salient_channels):
    B, C, H, W = x.shape
    CO = 4 * C
    N = B * H * W
    R = 512
    while N % R:
        R //= 2

    xt = jnp.transpose(x, (0, 2, 3, 1)).reshape(N, C)
    P = jax.nn.one_hot(salient_channels, CO, dtype=x.dtype)

    out2 = pl.pallas_call(
        _permute_body,
        grid=(N // R,),
        in_specs=[
            pl.BlockSpec((R, C), lambda i: (i, 0)),
            pl.BlockSpec((C, CO), lambda i: (0, 0)),
        ],
        out_specs=pl.BlockSpec((R, CO), lambda i: (i, 0)),
        out_shape=jax.ShapeDtypeStruct((N, CO), x.dtype),
    )(xt, P)
    return out2.reshape(B, H, W, CO).transpose(0, 3, 1, 2)


# speedup vs baseline: 1.7043x; 1.2591x over previous
import jax
import jax.numpy as jnp
from jax.experimental import pallas as pl


def _permute_body(x_ref, p_ref, o_ref):
    xv = x_ref[...]
    pv = p_ref[...]
    x0 = xv.astype(jnp.bfloat16).astype(jnp.float32)
    r1 = xv - x0
    x1 = r1.astype(jnp.bfloat16).astype(jnp.float32)
    x2 = r1 - x1

    def mm(a):
        return jnp.dot(a, pv, preferred_element_type=jnp.float32)

    o_ref[...] = mm(x0) + mm(x1) + mm(x2)


def kernel(x, salient_channels):
    B, C, H, W = x.shape
    CO = 4 * C
    N = B * H * W
    R = 1024
    while N % R:
        R //= 2

    xt = jnp.transpose(x, (0, 2, 3, 1)).reshape(N, C)
    P = jax.nn.one_hot(salient_channels, CO, dtype=x.dtype)

    out2 = pl.pallas_call(
        _permute_body,
        grid=(N // R,),
        in_specs=[
            pl.BlockSpec((R, C), lambda i: (i, 0)),
            pl.BlockSpec((C, CO), lambda i: (0, 0)),
        ],
        out_specs=pl.BlockSpec((R, CO), lambda i: (i, 0)),
        out_shape=jax.ShapeDtypeStruct((N, CO), x.dtype),
    )(xt, P)
    return out2.reshape(B, H, W, CO).transpose(0, 3, 1, 2)


# speedup vs baseline: 1.9895x; 1.1673x over previous
import jax
import jax.numpy as jnp
from jax.experimental import pallas as pl


def _permute_body(x_ref, p_ref, o_ref):
    xv = x_ref[...]
    pv = p_ref[...]
    x0 = xv.astype(jnp.bfloat16).astype(jnp.float32)
    r1 = xv - x0
    x1 = r1.astype(jnp.bfloat16).astype(jnp.float32)
    x2 = r1 - x1

    def mm(a):
        return jnp.dot(a, pv, preferred_element_type=jnp.float32)

    o_ref[...] = mm(x0) + mm(x1) + mm(x2)


def kernel(x, salient_channels):
    B, C, H, W = x.shape
    CO = 4 * C
    N = B * H * W
    R = 2048
    while N % R:
        R //= 2

    xt = jnp.transpose(x, (0, 2, 3, 1)).reshape(N, C)
    P = jax.nn.one_hot(salient_channels, CO, dtype=x.dtype)

    out2 = pl.pallas_call(
        _permute_body,
        grid=(N // R,),
        in_specs=[
            pl.BlockSpec((R, C), lambda i: (i, 0)),
            pl.BlockSpec((C, CO), lambda i: (0, 0)),
        ],
        out_specs=pl.BlockSpec((R, CO), lambda i: (i, 0)),
        out_shape=jax.ShapeDtypeStruct((N, CO), x.dtype),
    )(xt, P)
    return out2.reshape(B, H, W, CO).transpose(0, 3, 1, 2)
